# Initial kernel scaffold; baseline (speedup 1.0000x reference)
import functools

import jax
import jax.numpy as jnp
from jax import lax
from jax.experimental import pallas as pl
from jax.experimental.pallas import tpu as pltpu
from jax.experimental.pallas import tpu_sc as plsc

EMBED_DIM = 64
NC = 2
NS = 16
NW = NC * NS
TOTAL = 4096 * 50
B_PER_W = TOTAL // NW
CHUNK = 128
NCHUNK = B_PER_W // CHUNK

_mesh = plsc.VectorSubcoreMesh(core_axis_name="c", subcore_axis_name="s")


@functools.partial(
    pl.kernel,
    mesh=_mesh,
    out_type=jax.ShapeDtypeStruct((TOTAL, EMBED_DIM), jnp.float32),
    scratch_types=[
        pltpu.VMEM((NCHUNK, CHUNK), jnp.int32),
        pltpu.VMEM((2, CHUNK, EMBED_DIM), jnp.float32),
        pltpu.SemaphoreType.DMA,
        pltpu.SemaphoreType.DMA,
    ],
)
def _embed(src_hbm, table_hbm, out_hbm, idx_v, rows_v, gsem0, gsem1):
    wid = lax.axis_index("s") * NC + lax.axis_index("c")
    base = wid * B_PER_W

    pltpu.sync_copy(src_hbm.at[wid], idx_v)

    gsems = (gsem0, gsem1)

    def gather(c, buf):
        return pltpu.async_copy(
            table_hbm.at[idx_v.at[c]], rows_v.at[buf], gsems[buf]
        )

    def emit(c, buf):
        pltpu.sync_copy(rows_v.at[buf], out_hbm.at[pl.ds(base + c * CHUNK, CHUNK)])

    gather(0, 0).wait()

    @pl.loop(0, NCHUNK - 1, step=2)
    def _(g):
        nxt = gather(g + 1, 1)
        emit(g, 0)
        nxt.wait()
        nxt2 = gather(g + 2, 0)
        emit(g + 1, 1)
        nxt2.wait()

    emit(NCHUNK - 1, 0)


def kernel(src, table):
    src_w = src.astype(jnp.int32).reshape(NW, NCHUNK, CHUNK)
    out = _embed(src_w, table)
    return out.reshape(src.shape + (EMBED_DIM,))


# speedup vs baseline: 4.5461x; 4.5461x over previous
import functools

import jax
import jax.numpy as jnp
from jax import lax
from jax.experimental import pallas as pl
from jax.experimental.pallas import tpu as pltpu
from jax.experimental.pallas import tpu_sc as plsc

EMBED_DIM = 64
NC = 2
NS = 16
NW = NC * NS
TOTAL = 4096 * 50
B_PER_W = TOTAL // NW
CHUNK = 128
NCHUNK = B_PER_W // CHUNK

_mesh = plsc.VectorSubcoreMesh(core_axis_name="c", subcore_axis_name="s")


@functools.partial(
    pl.kernel,
    mesh=_mesh,
    out_type=jax.ShapeDtypeStruct((TOTAL, EMBED_DIM), jnp.float32),
    scratch_types=[
        pltpu.VMEM((NCHUNK, CHUNK), jnp.int32),
        pltpu.VMEM((2, CHUNK, EMBED_DIM), jnp.float32),
        pltpu.SemaphoreType.DMA,
        pltpu.SemaphoreType.DMA,
    ],
    compiler_params=pltpu.CompilerParams(use_tc_tiling_on_sc=False),
)
def _embed(src_hbm, table_hbm, out_hbm, idx_v, rows_v, gsem0, gsem1):
    wid = lax.axis_index("s") * NC + lax.axis_index("c")
    base = wid * B_PER_W

    pltpu.sync_copy(src_hbm.at[wid], idx_v)

    gsems = (gsem0, gsem1)

    def gather_start(c, buf):
        pltpu.async_copy(table_hbm.at[idx_v.at[c]], rows_v.at[buf], gsems[buf])

    def gather_wait(buf):
        pltpu.make_async_copy(
            table_hbm.at[pl.ds(0, CHUNK)], rows_v.at[buf], gsems[buf]
        ).wait()

    def emit(c, buf):
        pltpu.sync_copy(rows_v.at[buf], out_hbm.at[pl.ds(base + c * CHUNK, CHUNK)])

    gather_start(0, 0)

    @pl.loop(0, NCHUNK - 2, step=2)
    def _(g):
        gather_start(g + 1, 1)
        gather_wait(0)
        emit(g, 0)
        gather_start(g + 2, 0)
        gather_wait(1)
        emit(g + 1, 1)

    gather_start(NCHUNK - 1, 1)
    gather_wait(0)
    emit(NCHUNK - 2, 0)
    gather_wait(1)
    emit(NCHUNK - 1, 1)


def kernel(src, table):
    src_w = src.astype(jnp.int32).reshape(NW, NCHUNK, CHUNK)
    out = _embed(src_w, table)
    return out.reshape(src.shape + (EMBED_DIM,))


# speedup vs baseline: 4.6675x; 1.0267x over previous
import functools

import jax
import jax.numpy as jnp
from jax import lax
from jax.experimental import pallas as pl
from jax.experimental.pallas import tpu as pltpu
from jax.experimental.pallas import tpu_sc as plsc

EMBED_DIM = 64
NC = 2
NS = 16
NW = NC * NS
TOTAL = 4096 * 50
B_PER_W = TOTAL // NW
CHUNK = 800
NCHUNK = B_PER_W // CHUNK

_mesh = plsc.VectorSubcoreMesh(core_axis_name="c", subcore_axis_name="s")


@functools.partial(
    pl.kernel,
    mesh=_mesh,
    out_type=jax.ShapeDtypeStruct((TOTAL, EMBED_DIM), jnp.float32),
    scratch_types=[
        pltpu.VMEM((NCHUNK, CHUNK), jnp.int32),
        pltpu.VMEM((2, CHUNK, EMBED_DIM), jnp.float32),
        pltpu.SemaphoreType.DMA,
        pltpu.SemaphoreType.DMA,
    ],
    compiler_params=pltpu.CompilerParams(use_tc_tiling_on_sc=False),
)
def _embed(src_hbm, table_hbm, out_hbm, idx_v, rows_v, gsem0, gsem1):
    wid = lax.axis_index("s") * NC + lax.axis_index("c")
    base = wid * B_PER_W

    pltpu.sync_copy(src_hbm.at[wid], idx_v)

    gsems = (gsem0, gsem1)

    def gather_start(c, buf):
        pltpu.async_copy(table_hbm.at[idx_v.at[c]], rows_v.at[buf], gsems[buf])

    def gather_wait(buf):
        pltpu.make_async_copy(
            table_hbm.at[pl.ds(0, CHUNK)], rows_v.at[buf], gsems[buf]
        ).wait()

    def emit(c, buf):
        pltpu.sync_copy(rows_v.at[buf], out_hbm.at[pl.ds(base + c * CHUNK, CHUNK)])

    gather_start(0, 0)

    @pl.loop(0, NCHUNK - 2, step=2)
    def _(g):
        gather_start(g + 1, 1)
        gather_wait(0)
        emit(g, 0)
        gather_start(g + 2, 0)
        gather_wait(1)
        emit(g + 1, 1)

    gather_start(NCHUNK - 1, 1)
    gather_wait(0)
    emit(NCHUNK - 2, 0)
    gather_wait(1)
    emit(NCHUNK - 1, 1)


def kernel(src, table):
    src_w = src.astype(jnp.int32).reshape(NW, NCHUNK, CHUNK)
    out = _embed(src_w, table)
    return out.reshape(src.shape + (EMBED_DIM,))


# speedup vs baseline: 4.6797x; 1.0026x over previous
import functools

import jax
import jax.numpy as jnp
from jax import lax
from jax.experimental import pallas as pl
from jax.experimental.pallas import tpu as pltpu
from jax.experimental.pallas import tpu_sc as plsc

EMBED_DIM = 64
NC = 2
NS = 16
NW = NC * NS
ROWS, SEQ = 4096, 50
R_PER_W = ROWS // NW
G = 8
NGROUP = R_PER_W // G

_mesh = plsc.VectorSubcoreMesh(core_axis_name="c", subcore_axis_name="s")


@functools.partial(
    pl.kernel,
    mesh=_mesh,
    out_type=jax.ShapeDtypeStruct((ROWS, SEQ, EMBED_DIM), jnp.float32),
    scratch_types=[
        pltpu.VMEM((R_PER_W, SEQ), jnp.int32),
        pltpu.VMEM((2, G, SEQ, EMBED_DIM), jnp.float32),
        pltpu.SemaphoreType.DMA,
        pltpu.SemaphoreType.DMA,
    ],
    compiler_params=pltpu.CompilerParams(use_tc_tiling_on_sc=False),
)
def _embed(src_hbm, table_hbm, out_hbm, idx_v, rows_v, gsem0, gsem1):
    wid = lax.axis_index("s") * NC + lax.axis_index("c")
    rbase = wid * R_PER_W

    pltpu.sync_copy(src_hbm.at[pl.ds(rbase, R_PER_W)], idx_v)

    gsems = (gsem0, gsem1)

    def group_start(c, buf):
        for j in range(G):
            pltpu.async_copy(
                table_hbm.at[idx_v.at[c * G + j]], rows_v.at[buf, j], gsems[buf]
            )

    def group_wait(buf):
        pltpu.make_async_copy(
            out_hbm.at[pl.ds(0, G)], rows_v.at[buf], gsems[buf]
        ).wait()

    def emit(c, buf):
        pltpu.sync_copy(rows_v.at[buf], out_hbm.at[pl.ds(rbase + c * G, G)])

    group_start(0, 0)

    @pl.loop(0, NGROUP - 2, step=2)
    def _(g):
        group_start(g + 1, 1)
        group_wait(0)
        emit(g, 0)
        group_start(g + 2, 0)
        group_wait(1)
        emit(g + 1, 1)

    group_start(NGROUP - 1, 1)
    group_wait(0)
    emit(NGROUP - 2, 0)
    group_wait(1)
    emit(NGROUP - 1, 1)


def kernel(src, table):
    return _embed(src.astype(jnp.int32), table)
